# baseline (speedup 1.0000x reference)
import jax
import jax.numpy as jnp
from jax import lax
from jax.experimental import pallas as pl
from jax.experimental.pallas import tpu as pltpu
from jax.experimental.pallas import tpu_sc as plsc

B, C_HALF, H, W = 32, 192, 64, 64
C_TOTAL = 2 * C_HALF
HW = H * W
SUB, LANE = 32, 128


def _tc_body(fwd_ref, x1_ref, x2_ref, o1_ref):
    for j in range(C_HALF):
        s = fwd_ref[j]

        @pl.when(s < C_HALF)
        def _(s=s, j=j):
            o1_ref[0, pl.ds(j, 1)] = x1_ref[0, pl.ds(s, 1)]

        @pl.when(s >= C_HALF)
        def _(s=s, j=j):
            o1_ref[0, pl.ds(j, 1)] = x2_ref[0, pl.ds(s - C_HALF, 1)]


def _tc_half(x1r, x2r, fwd_idxs):
    block = (1, C_HALF, SUB, LANE)
    grid_spec = pltpu.PrefetchScalarGridSpec(
        num_scalar_prefetch=1,
        grid=(B,),
        in_specs=[
            pl.BlockSpec(block, lambda b, f: (b, 0, 0, 0)),
            pl.BlockSpec(block, lambda b, f: (b, 0, 0, 0)),
        ],
        out_specs=pl.BlockSpec(block, lambda b, f: (b, 0, 0, 0)),
    )
    return pl.pallas_call(
        _tc_body,
        grid_spec=grid_spec,
        out_shape=jax.ShapeDtypeStruct((B, C_HALF, SUB, LANE), jnp.float32),
    )(fwd_idxs[:C_HALF].astype(jnp.int32), x1r, x2r)


NC = 2
NS = 16
NWORK = NC * NS
CH_PER_W = C_HALF // NWORK
PARTS = 4
BCH = B // PARTS
NCHUNK = CH_PER_W * PARTS
NBUF = 3


def _sc_body(fwdw_hbm, x1_hbm, x2_hbm, out2_hbm, myfwd_v, buf, in_sem, out_sem):
    c = lax.axis_index("c")
    sid = lax.axis_index("s")
    wid = c * NS + sid
    pltpu.sync_copy(fwdw_hbm.at[pl.ds(wid * 16, 16)], myfwd_v)
    srcs = myfwd_v[...]

    def start_gather(t, slot):
        k, part = divmod(t, PARTS)
        b0 = part * BCH
        s = srcs[k]

        @pl.when(s < C_HALF)
        def _():
            pltpu.async_copy(
                x1_hbm.at[pl.ds(b0, BCH), pl.ds(s, 1)],
                buf.at[slot], in_sem.at[slot])

        @pl.when(s >= C_HALF)
        def _():
            pltpu.async_copy(
                x2_hbm.at[pl.ds(b0, BCH), pl.ds(s - C_HALF, 1)],
                buf.at[slot], in_sem.at[slot])

    def wait_in(slot):
        pltpu.make_async_copy(
            x1_hbm.at[pl.ds(0, BCH), pl.ds(0, 1)], buf.at[slot],
            in_sem.at[slot]).wait()

    def wait_out(slot):
        pltpu.make_async_copy(
            buf.at[slot], out2_hbm.at[pl.ds(0, BCH), pl.ds(0, 1)],
            out_sem.at[slot]).wait()

    def start_scatter(u, slot):
        k, part = divmod(u, PARTS)
        b0 = part * BCH
        chl = wid * CH_PER_W + k
        pltpu.async_copy(
            buf.at[slot], out2_hbm.at[pl.ds(b0, BCH), pl.ds(chl, 1)],
            out_sem.at[slot])

    for p in range(NBUF - 1):
        start_gather(p, p)
    for t in range(NCHUNK):
        nxt = t + NBUF - 1
        if nxt < NCHUNK:
            if nxt >= NBUF:
                wait_out(nxt % NBUF)
            start_gather(nxt, nxt % NBUF)
        wait_in(t % NBUF)
        start_scatter(t, t % NBUF)
    for p in range(NBUF):
        wait_out(p)


def _sc_half(x1f, x2f, fwd_idxs):
    out_t = jax.ShapeDtypeStruct((B, C_HALF, HW), jnp.float32)
    fwdw = jnp.pad(
        fwd_idxs[C_HALF:].astype(jnp.int32).reshape(NWORK, CH_PER_W),
        ((0, 0), (0, 16 - CH_PER_W)),
    ).reshape(-1)
    f = pl.kernel(
        _sc_body,
        out_type=out_t,
        mesh=plsc.VectorSubcoreMesh(core_axis_name="c", subcore_axis_name="s"),
        scratch_types=[
            pltpu.VMEM((16,), jnp.int32),
            pltpu.VMEM((NBUF, BCH, 1, HW), jnp.float32),
            pltpu.SemaphoreType.DMA((NBUF,)),
            pltpu.SemaphoreType.DMA((NBUF,)),
        ],
    )
    return f(fwdw, x1f, x2f)


def kernel(x1, x2, sldj_x, fwd_idxs):
    out1 = _tc_half(
        x1.reshape(B, C_HALF, SUB, LANE),
        x2.reshape(B, C_HALF, SUB, LANE),
        fwd_idxs,
    )
    out2 = _sc_half(
        x1.reshape(B, C_HALF, HW),
        x2.reshape(B, C_HALF, HW),
        fwd_idxs,
    )
    return (
        out1.reshape(B, C_HALF, H, W),
        out2.reshape(B, C_HALF, H, W),
        sldj_x,
    )


# speedup vs baseline: 1.4206x; 1.4206x over previous
import jax
import jax.numpy as jnp
from jax import lax
from jax.experimental import pallas as pl
from jax.experimental.pallas import tpu as pltpu
from jax.experimental.pallas import tpu_sc as plsc

B, C_HALF, H, W = 32, 192, 64, 64
C_TOTAL = 2 * C_HALF
HW = H * W

NC = 2
NS = 16
NWORK = NC * NS
CH_PER_W = C_HALF // NWORK
PARTS = 4
BCH = B // PARTS
NCHUNK = 2 * CH_PER_W * PARTS
NBUF = 3


def _sc_body(fwdw_hbm, x1_hbm, x2_hbm, out1_hbm, out2_hbm,
             myfwd_v, buf, in_sem, out_sem):
    c = lax.axis_index("c")
    sid = lax.axis_index("s")
    wid = c * NS + sid
    pltpu.sync_copy(fwdw_hbm.at[pl.ds(wid * 16, 16)], myfwd_v)
    srcs = myfwd_v[...]

    def start_gather(t, slot):
        k, part = divmod(t, PARTS)
        b0 = part * BCH
        s = srcs[k]

        @pl.when(s < C_HALF)
        def _():
            pltpu.async_copy(
                x1_hbm.at[pl.ds(b0, BCH), pl.ds(s, 1)],
                buf.at[slot], in_sem.at[slot])

        @pl.when(s >= C_HALF)
        def _():
            pltpu.async_copy(
                x2_hbm.at[pl.ds(b0, BCH), pl.ds(s - C_HALF, 1)],
                buf.at[slot], in_sem.at[slot])

    def wait_in(slot):
        pltpu.make_async_copy(
            x1_hbm.at[pl.ds(0, BCH), pl.ds(0, 1)], buf.at[slot],
            in_sem.at[slot]).wait()

    def wait_out(slot):
        pltpu.make_async_copy(
            buf.at[slot], out1_hbm.at[pl.ds(0, BCH), pl.ds(0, 1)],
            out_sem.at[slot]).wait()

    def start_scatter(u, slot):
        k, part = divmod(u, PARTS)
        b0 = part * BCH
        out_ref = out1_hbm if k < CH_PER_W else out2_hbm
        chl = wid * CH_PER_W + (k % CH_PER_W)
        pltpu.async_copy(
            buf.at[slot], out_ref.at[pl.ds(b0, BCH), pl.ds(chl, 1)],
            out_sem.at[slot])

    for p in range(NBUF - 1):
        start_gather(p, p)
    for t in range(NCHUNK):
        nxt = t + NBUF - 1
        if nxt < NCHUNK:
            if nxt >= NBUF:
                wait_out(nxt % NBUF)
            start_gather(nxt, nxt % NBUF)
        wait_in(t % NBUF)
        start_scatter(t, t % NBUF)
    for p in range(NBUF):
        wait_out(p)


def kernel(x1, x2, sldj_x, fwd_idxs):
    x1r = x1.reshape(B, C_HALF, HW)
    x2r = x2.reshape(B, C_HALF, HW)
    out_t = jax.ShapeDtypeStruct((B, C_HALF, HW), jnp.float32)

    fwd_i = fwd_idxs.astype(jnp.int32)
    per_w = jnp.concatenate(
        [
            fwd_i[:C_HALF].reshape(NWORK, CH_PER_W),
            fwd_i[C_HALF:].reshape(NWORK, CH_PER_W),
        ],
        axis=1,
    )
    fwdw = jnp.pad(per_w, ((0, 0), (0, 16 - 2 * CH_PER_W))).reshape(-1)

    f = pl.kernel(
        _sc_body,
        out_type=[out_t, out_t],
        mesh=plsc.VectorSubcoreMesh(core_axis_name="c", subcore_axis_name="s"),
        scratch_types=[
            pltpu.VMEM((16,), jnp.int32),
            pltpu.VMEM((NBUF, BCH, 1, HW), jnp.float32),
            pltpu.SemaphoreType.DMA((NBUF,)),
            pltpu.SemaphoreType.DMA((NBUF,)),
        ],
    )
    out1, out2 = f(fwdw, x1r, x2r)
    return (
        out1.reshape(B, C_HALF, H, W),
        out2.reshape(B, C_HALF, H, W),
        sldj_x,
    )
